# Initial kernel scaffold; baseline (speedup 1.0000x reference)
import jax
import jax.numpy as jnp
from jax.experimental import pallas as pl


def kernel(x, edge_index, W0, b0, W1, b1, W2, b2, W3, b3, RW0, Rb0, RW1, Rb1):
    raise NotImplementedError("write your pallas kernel here")


# speedup vs baseline: 5.0971x; 5.0971x over previous
import functools

import jax
import jax.numpy as jnp
from jax import lax
from jax.experimental import pallas as pl
from jax.experimental.pallas import tpu as pltpu
from jax.experimental.pallas import tpu_sc as plsc

N = 10000
E = 160000
NFEAT = 256
NHID = 256
NCLASS = 40
DH = 128
DC = 128

NSC = 2
NSUB = 16
HALF = N // NSC
NPAD = 5120
ROWS_PER_TILE = NPAD // NSUB
DUMMY = NPAD - 1
ET = E // NSUB
K = 80
NCHUNK = ET // K
LANES = 16

R = 1000
G = N // R
RB = HALF // R


@functools.cache
def _get_mesh():
    return plsc.VectorSubcoreMesh(
        core_axis_name="c", subcore_axis_name="s", num_cores=NSC, num_subcores=NSUB
    )


def _sc_prep_body(dst_hbm, dstloc_hbm, degacc_hbm, dbuf, dlbuf, ones, zbuf, acc):
    c = lax.axis_index("c")
    s = lax.axis_index("s")

    def fill_ones(j, _):
        ones[j, :] = jnp.full((LANES,), 1.0, jnp.float32)
        zbuf[j, :] = jnp.zeros((LANES,), jnp.float32)
        return 0
    lax.fori_loop(0, K, fill_ones, 0)

    def zero_acc(q, _):
        pltpu.sync_copy(zbuf, acc.at[pl.ds(s * ROWS_PER_TILE + q * K, K), :])
        return 0
    lax.fori_loop(0, ROWS_PER_TILE // K, zero_acc, 0)
    plsc.subcore_barrier()

    def chunk(i, _):
        base = s * ET + i * K
        pltpu.sync_copy(dst_hbm.at[pl.ds(base, K)], dbuf)

        def vec(j, _):
            v = dbuf[pl.ds(j * LANES, LANES)]
            l = v - c * HALF
            ok = (l >= 0) & (l < HALF)
            dlbuf[pl.ds(j * LANES, LANES)] = jnp.where(ok, l, DUMMY)
            return 0
        lax.fori_loop(0, K // LANES, vec, 0)

        pltpu.sync_copy(dlbuf, dstloc_hbm.at[pl.ds(c * E + base, K)])
        pltpu.sync_copy(ones, acc.at[dlbuf], add=True)
        return 0
    lax.fori_loop(0, NCHUNK, chunk, 0)
    plsc.subcore_barrier()

    pltpu.sync_copy(
        acc.at[pl.ds(s * ROWS_PER_TILE, ROWS_PER_TILE), :],
        degacc_hbm.at[c, pl.ds(s * ROWS_PER_TILE, ROWS_PER_TILE), :],
    )


@functools.cache
def _get_sc_prep():
    return pl.kernel(
        _sc_prep_body,
        out_type=[
            jax.ShapeDtypeStruct((NSC * E,), jnp.int32),
            jax.ShapeDtypeStruct((NSC, NPAD, LANES), jnp.float32),
        ],
        mesh=_get_mesh(),
        scratch_types=[
            pltpu.VMEM((K,), jnp.int32),
            pltpu.VMEM((K,), jnp.int32),
            pltpu.VMEM((K, LANES), jnp.float32),
            pltpu.VMEM((K, LANES), jnp.float32),
            pltpu.VMEM_SHARED((NPAD, LANES), jnp.float32),
        ],
    )


def _sc_edge2_body(ua_hbm, ub_hbm, src_hbm, dstloc_hbm, sa_hbm, sb_hbm,
                   sbuf, dlb, gba, gbb, zrow, acca, accb, sema, semb):
    c = lax.axis_index("c")
    s = lax.axis_index("s")

    def zfill(j, _):
        def zlane(kk, _):
            zrow[j, pl.ds(kk * LANES, LANES)] = jnp.zeros((LANES,), jnp.float32)
            return 0
        lax.fori_loop(0, DH // LANES, zlane, 0)
        return 0
    lax.fori_loop(0, K, zfill, 0)

    def zero_acc(q, _):
        pltpu.sync_copy(zrow, acca.at[pl.ds(s * ROWS_PER_TILE + q * K, K), :])
        pltpu.sync_copy(zrow, accb.at[pl.ds(s * ROWS_PER_TILE + q * K, K), :])
        return 0
    lax.fori_loop(0, ROWS_PER_TILE // K, zero_acc, 0)
    plsc.subcore_barrier()

    def chunk(i, _):
        base = s * ET + i * K
        pltpu.sync_copy(src_hbm.at[pl.ds(base, K)], sbuf)
        pltpu.sync_copy(dstloc_hbm.at[pl.ds(c * E + base, K)], dlb)
        cpa = pltpu.async_copy(ua_hbm.at[sbuf], gba, sema)
        cpb = pltpu.async_copy(ub_hbm.at[sbuf], gbb, semb)
        cpa.wait()
        pltpu.sync_copy(gba, acca.at[dlb], add=True)
        cpb.wait()
        pltpu.sync_copy(gbb, accb.at[dlb], add=True)
        return 0
    lax.fori_loop(0, NCHUNK, chunk, 0)
    plsc.subcore_barrier()

    pltpu.sync_copy(
        acca.at[pl.ds(s * ROWS_PER_TILE, ROWS_PER_TILE), :],
        sa_hbm.at[c, pl.ds(s * ROWS_PER_TILE, ROWS_PER_TILE), :],
    )
    pltpu.sync_copy(
        accb.at[pl.ds(s * ROWS_PER_TILE, ROWS_PER_TILE), :],
        sb_hbm.at[c, pl.ds(s * ROWS_PER_TILE, ROWS_PER_TILE), :],
    )


@functools.cache
def _get_sc_edge2():
    return pl.kernel(
        _sc_edge2_body,
        out_type=[
            jax.ShapeDtypeStruct((NSC, NPAD, DH), jnp.float32),
            jax.ShapeDtypeStruct((NSC, NPAD, DH), jnp.float32),
        ],
        mesh=_get_mesh(),
        scratch_types=[
            pltpu.VMEM((K,), jnp.int32),
            pltpu.VMEM((K,), jnp.int32),
            pltpu.VMEM((K, DH), jnp.float32),
            pltpu.VMEM((K, DH), jnp.float32),
            pltpu.VMEM((K, DH), jnp.float32),
            pltpu.VMEM_SHARED((NPAD, DH), jnp.float32),
            pltpu.VMEM_SHARED((NPAD, DH), jnp.float32),
            pltpu.SemaphoreType.DMA,
            pltpu.SemaphoreType.DMA,
        ],
    )


def _sc_edge1_body(u_hbm, src_hbm, dstloc_hbm, s_hbm, sbuf, dlb, gbuf, zrow, acc, sem):
    c = lax.axis_index("c")
    s = lax.axis_index("s")

    def zfill(j, _):
        def zlane(kk, _):
            zrow[j, pl.ds(kk * LANES, LANES)] = jnp.zeros((LANES,), jnp.float32)
            return 0
        lax.fori_loop(0, DC // LANES, zlane, 0)
        return 0
    lax.fori_loop(0, K, zfill, 0)

    def zero_acc(q, _):
        pltpu.sync_copy(zrow, acc.at[pl.ds(s * ROWS_PER_TILE + q * K, K), :])
        return 0
    lax.fori_loop(0, ROWS_PER_TILE // K, zero_acc, 0)
    plsc.subcore_barrier()

    def chunk(i, _):
        base = s * ET + i * K
        pltpu.sync_copy(src_hbm.at[pl.ds(base, K)], sbuf)
        pltpu.sync_copy(dstloc_hbm.at[pl.ds(c * E + base, K)], dlb)
        pltpu.async_copy(u_hbm.at[sbuf], gbuf, sem).wait()
        pltpu.sync_copy(gbuf, acc.at[dlb], add=True)
        return 0
    lax.fori_loop(0, NCHUNK, chunk, 0)
    plsc.subcore_barrier()

    pltpu.sync_copy(
        acc.at[pl.ds(s * ROWS_PER_TILE, ROWS_PER_TILE), :],
        s_hbm.at[c, pl.ds(s * ROWS_PER_TILE, ROWS_PER_TILE), :],
    )


@functools.cache
def _get_sc_edge1():
    return pl.kernel(
        _sc_edge1_body,
        out_type=jax.ShapeDtypeStruct((NSC, NPAD, DC), jnp.float32),
        mesh=_get_mesh(),
        scratch_types=[
            pltpu.VMEM((K,), jnp.int32),
            pltpu.VMEM((K,), jnp.int32),
            pltpu.VMEM((K, DC), jnp.float32),
            pltpu.VMEM((K, DC), jnp.float32),
            pltpu.VMEM_SHARED((NPAD, DC), jnp.float32),
            pltpu.SemaphoreType.DMA,
        ],
    )


def _half_map(i):
    return (i // RB, i % RB, 0)


def _row_map(i):
    return (i, 0)


def _rep_map(i):
    return (0, 0)


def _mm0_body(x_ref, w_ref, rw_ref, deg_ref, ua_ref, ub_ref, r_ref, dinv_ref):
    dinv = lax.rsqrt(deg_ref[0, :, 0:1] + 1.0)
    xb = x_ref[...]
    u = jnp.dot(xb, w_ref[...], preferred_element_type=jnp.float32) * dinv
    ua_ref[...] = u[:, :DH]
    ub_ref[...] = u[:, DH:]
    r_ref[...] = jnp.dot(xb, rw_ref[...], preferred_element_type=jnp.float32)
    dinv_ref[...] = dinv


_mm0 = pl.pallas_call(
    _mm0_body,
    grid=(G,),
    in_specs=[
        pl.BlockSpec((R, NFEAT), _row_map),
        pl.BlockSpec((NFEAT, NHID), _rep_map),
        pl.BlockSpec((NFEAT, NHID), _rep_map),
        pl.BlockSpec((1, R, LANES), _half_map),
    ],
    out_specs=[
        pl.BlockSpec((R, DH), _row_map),
        pl.BlockSpec((R, DH), _row_map),
        pl.BlockSpec((R, NHID), _row_map),
        pl.BlockSpec((R, 1), _row_map),
    ],
    out_shape=[
        jax.ShapeDtypeStruct((N, DH), jnp.float32),
        jax.ShapeDtypeStruct((N, DH), jnp.float32),
        jax.ShapeDtypeStruct((N, NHID), jnp.float32),
        jax.ShapeDtypeStruct((N, 1), jnp.float32),
    ],
)


def _mm_body(h_ref, w_ref, dinv_ref, ua_ref, ub_ref):
    u = jnp.dot(
        h_ref[...], w_ref[...], preferred_element_type=jnp.float32
    ) * dinv_ref[...]
    ua_ref[...] = u[:, :DH]
    ub_ref[...] = u[:, DH:]


_mm_mid = pl.pallas_call(
    _mm_body,
    grid=(G,),
    in_specs=[
        pl.BlockSpec((R, NHID), _row_map),
        pl.BlockSpec((NHID, NHID), _rep_map),
        pl.BlockSpec((R, 1), _row_map),
    ],
    out_specs=[
        pl.BlockSpec((R, DH), _row_map),
        pl.BlockSpec((R, DH), _row_map),
    ],
    out_shape=[
        jax.ShapeDtypeStruct((N, DH), jnp.float32),
        jax.ShapeDtypeStruct((N, DH), jnp.float32),
    ],
)


def _mm3_body(h_ref, w_ref, rw_ref, dinv_ref, u_ref, r_ref):
    hb = h_ref[...]
    u_ref[...] = jnp.dot(hb, w_ref[...], preferred_element_type=jnp.float32) * dinv_ref[...]
    r_ref[...] = jnp.dot(hb, rw_ref[...], preferred_element_type=jnp.float32)


_mm3 = pl.pallas_call(
    _mm3_body,
    grid=(G,),
    in_specs=[
        pl.BlockSpec((R, NHID), _row_map),
        pl.BlockSpec((NHID, DC), _rep_map),
        pl.BlockSpec((NHID, DC), _rep_map),
        pl.BlockSpec((R, 1), _row_map),
    ],
    out_specs=[
        pl.BlockSpec((R, DC), _row_map),
        pl.BlockSpec((R, DC), _row_map),
    ],
    out_shape=[
        jax.ShapeDtypeStruct((N, DC), jnp.float32),
        jax.ShapeDtypeStruct((N, DC), jnp.float32),
    ],
)


def _ep0_body(sa_ref, sb_ref, ua_ref, ub_ref, r_ref, dinv_ref, b_ref, rb_ref, o_ref):
    s = jnp.concatenate([sa_ref[0], sb_ref[0]], axis=1)
    u = jnp.concatenate([ua_ref[...], ub_ref[...]], axis=1)
    o_ref[...] = jax.nn.relu(
        dinv_ref[...] * (s + u) + r_ref[...] + b_ref[...] + rb_ref[...]
    )


_ep0 = pl.pallas_call(
    _ep0_body,
    grid=(G,),
    in_specs=[
        pl.BlockSpec((1, R, DH), _half_map),
        pl.BlockSpec((1, R, DH), _half_map),
        pl.BlockSpec((R, DH), _row_map),
        pl.BlockSpec((R, DH), _row_map),
        pl.BlockSpec((R, NHID), _row_map),
        pl.BlockSpec((R, 1), _row_map),
        pl.BlockSpec((1, NHID), _rep_map),
        pl.BlockSpec((1, NHID), _rep_map),
    ],
    out_specs=pl.BlockSpec((R, NHID), _row_map),
    out_shape=jax.ShapeDtypeStruct((N, NHID), jnp.float32),
)


def _ep_mid_body(sa_ref, sb_ref, ua_ref, ub_ref, h_ref, dinv_ref, b_ref, o_ref):
    s = jnp.concatenate([sa_ref[0], sb_ref[0]], axis=1)
    u = jnp.concatenate([ua_ref[...], ub_ref[...]], axis=1)
    o_ref[...] = jax.nn.relu(
        dinv_ref[...] * (s + u) + h_ref[...] + b_ref[...]
    )


_ep_mid = pl.pallas_call(
    _ep_mid_body,
    grid=(G,),
    in_specs=[
        pl.BlockSpec((1, R, DH), _half_map),
        pl.BlockSpec((1, R, DH), _half_map),
        pl.BlockSpec((R, DH), _row_map),
        pl.BlockSpec((R, DH), _row_map),
        pl.BlockSpec((R, NHID), _row_map),
        pl.BlockSpec((R, 1), _row_map),
        pl.BlockSpec((1, NHID), _rep_map),
    ],
    out_specs=pl.BlockSpec((R, NHID), _row_map),
    out_shape=jax.ShapeDtypeStruct((N, NHID), jnp.float32),
)


def _ep3_body(s_ref, u_ref, r_ref, dinv_ref, b_ref, rb_ref, o_ref):
    z = dinv_ref[...] * (s_ref[0] + u_ref[...]) + r_ref[...] + b_ref[...] + rb_ref[...]
    col = lax.broadcasted_iota(jnp.int32, (R, DC), 1)
    z = jnp.where(col < NCLASS, z, -1e30)
    m = jnp.max(z, axis=1, keepdims=True)
    lse = jnp.log(jnp.sum(jnp.exp(z - m), axis=1, keepdims=True)) + m
    o_ref[...] = z - lse


_ep3 = pl.pallas_call(
    _ep3_body,
    grid=(G,),
    in_specs=[
        pl.BlockSpec((1, R, DC), _half_map),
        pl.BlockSpec((R, DC), _row_map),
        pl.BlockSpec((R, DC), _row_map),
        pl.BlockSpec((R, 1), _row_map),
        pl.BlockSpec((1, DC), _rep_map),
        pl.BlockSpec((1, DC), _rep_map),
    ],
    out_specs=pl.BlockSpec((R, DC), _row_map),
    out_shape=jax.ShapeDtypeStruct((N, DC), jnp.float32),
)


def kernel(x, edge_index, W0, b0, W1, b1, W2, b2, W3, b3, RW0, Rb0, RW1, Rb1):
    ei = edge_index.astype(jnp.int32)
    src, dst = ei[0], ei[1]

    dstloc, degacc = _get_sc_prep()(dst)

    def pad_c(a):
        return jnp.zeros(a.shape[:-1] + (DC,), jnp.float32).at[..., :NCLASS].set(a)

    W3p, RW1p = pad_c(W3), pad_c(RW1)
    b3p, Rb1p = pad_c(b3).reshape(1, DC), pad_c(Rb1).reshape(1, DC)
    b0r, Rb0r = b0.reshape(1, NHID), Rb0.reshape(1, NHID)
    b1r, b2r = b1.reshape(1, NHID), b2.reshape(1, NHID)

    sc2 = _get_sc_edge2()
    sc1 = _get_sc_edge1()

    u0a, u0b, r0, dinv = _mm0(x, W0, RW0, degacc)
    s0a, s0b = sc2(u0a, u0b, src, dstloc)
    h1 = _ep0(s0a, s0b, u0a, u0b, r0, dinv, b0r, Rb0r)

    u1a, u1b = _mm_mid(h1, W1, dinv)
    s1a, s1b = sc2(u1a, u1b, src, dstloc)
    h2 = _ep_mid(s1a, s1b, u1a, u1b, h1, dinv, b1r)

    u2a, u2b = _mm_mid(h2, W2, dinv)
    s2a, s2b = sc2(u2a, u2b, src, dstloc)
    h3 = _ep_mid(s2a, s2b, u2a, u2b, h2, dinv, b2r)

    u3, r3 = _mm3(h3, W3p, RW1p, dinv)
    s3 = sc1(u3, src, dstloc)
    outp = _ep3(s3, u3, r3, dinv, b3p, Rb1p)
    return outp[:, :NCLASS]


# speedup vs baseline: 9.9352x; 1.9492x over previous
import functools

import jax
import jax.numpy as jnp
from jax import lax
from jax.experimental import pallas as pl
from jax.experimental.pallas import tpu as pltpu
from jax.experimental.pallas import tpu_sc as plsc

N = 10000
E = 160000
NFEAT = 256
NHID = 256
NCLASS = 40
DH = 128
DC = 128

NSC = 2
NSUB = 16
EPSC = E // NSC
K = 80
ETS = 5120
EPAD = NSUB * ETS
PADN = EPAD - EPSC
NCHUNK = ETS // K
DEPTH = 4
LANES = 16
ZR = 32

NPF = 10240
ROWS_OUT = NPF // NSUB

R = 1000
G = N // R


@functools.cache
def _get_mesh():
    return plsc.VectorSubcoreMesh(
        core_axis_name="c", subcore_axis_name="s", num_cores=NSC, num_subcores=NSUB
    )


def _sc_prep_body(dstp_hbm, degacc_hbm, dbuf, ones, zbuf, acc):
    c = lax.axis_index("c")
    s = lax.axis_index("s")

    def fill_ones(j, _):
        ones[j, :] = jnp.full((LANES,), 1.0, jnp.float32)
        zbuf[j, :] = jnp.zeros((LANES,), jnp.float32)
        return 0
    lax.fori_loop(0, K, fill_ones, 0)

    for q in range(ROWS_OUT // K):
        pltpu.sync_copy(zbuf, acc.at[pl.ds(s * ROWS_OUT + q * K, K)])
    plsc.subcore_barrier()

    def chunk(i, _):
        base = c * EPAD + s * ETS + i * K
        pltpu.sync_copy(dstp_hbm.at[pl.ds(base, K)], dbuf)
        pltpu.sync_copy(ones, acc.at[dbuf], add=True)
        return 0
    lax.fori_loop(0, NCHUNK, chunk, 0)
    plsc.subcore_barrier()

    pltpu.sync_copy(
        acc.at[pl.ds(s * ROWS_OUT, ROWS_OUT)],
        degacc_hbm.at[c, pl.ds(s * ROWS_OUT, ROWS_OUT)],
    )


@functools.cache
def _get_sc_prep():
    return pl.kernel(
        _sc_prep_body,
        out_type=jax.ShapeDtypeStruct((NSC, NPF, LANES), jnp.float32),
        mesh=_get_mesh(),
        scratch_types=[
            pltpu.VMEM((K,), jnp.int32),
            pltpu.VMEM((K, LANES), jnp.float32),
            pltpu.VMEM((K, LANES), jnp.float32),
            pltpu.VMEM_SHARED((NPF, LANES), jnp.float32),
        ],
    )


def _zfill(zrow):
    def zf(j, _):
        for kk in range(DH // LANES):
            zrow[j, pl.ds(kk * LANES, LANES)] = jnp.zeros((LANES,), jnp.float32)
        return 0
    lax.fori_loop(0, ZR, zf, 0)


def _edge_phase(u_hbm, out_hbm, srcp_hbm, dstp_hbm, ibs, dbs, gbs, zrow, acc,
                sems, c, s):
    def zero(q, _):
        pltpu.sync_copy(zrow, acc.at[pl.ds(s * ROWS_OUT + q * ZR, ZR)])
        return 0
    lax.fori_loop(0, ROWS_OUT // ZR, zero, 0)
    plsc.subcore_barrier()

    def quad(i, _):
        e0 = c * EPAD + s * ETS + i * (DEPTH * K)
        cps = []
        for j in range(DEPTH):
            pltpu.sync_copy(srcp_hbm.at[pl.ds(e0 + j * K, K)], ibs[j])
            pltpu.sync_copy(dstp_hbm.at[pl.ds(e0 + j * K, K)], dbs[j])
            cps.append(pltpu.async_copy(u_hbm.at[ibs[j]], gbs[j], sems[j]))
        for j in range(DEPTH):
            cps[j].wait()
            pltpu.sync_copy(gbs[j], acc.at[dbs[j]], add=True)
        return 0
    lax.fori_loop(0, NCHUNK // DEPTH, quad, 0)
    plsc.subcore_barrier()

    pltpu.sync_copy(
        acc.at[pl.ds(s * ROWS_OUT, ROWS_OUT)],
        out_hbm.at[c, pl.ds(s * ROWS_OUT, ROWS_OUT)],
    )


def _sc_edge_body(u_hbm, srcp_hbm, dstp_hbm, s_hbm,
                  ib0, ib1, ib2, ib3, db0, db1, db2, db3,
                  gb0, gb1, gb2, gb3, zrow, acc, sm0, sm1, sm2, sm3):
    c = lax.axis_index("c")
    s = lax.axis_index("s")
    ibs, dbs = (ib0, ib1, ib2, ib3), (db0, db1, db2, db3)
    gbs, sems = (gb0, gb1, gb2, gb3), (sm0, sm1, sm2, sm3)
    _zfill(zrow)
    _edge_phase(u_hbm, s_hbm, srcp_hbm, dstp_hbm, ibs, dbs, gbs, zrow, acc, sems, c, s)


def _edge_scratch():
    return (
        [pltpu.VMEM((K,), jnp.int32)] * 8
        + [pltpu.VMEM((K, DH), jnp.float32)] * 4
        + [pltpu.VMEM((ZR, DH), jnp.float32)]
        + [pltpu.VMEM_SHARED((NPF, DH), jnp.float32)]
        + [pltpu.SemaphoreType.DMA] * 4
    )


@functools.cache
def _get_sc_edge():
    return pl.kernel(
        _sc_edge_body,
        out_type=jax.ShapeDtypeStruct((NSC, NPF, DH), jnp.float32),
        mesh=_get_mesh(),
        scratch_types=_edge_scratch(),
    )


def _part_map(i):
    return (0, i, 0)


def _row_map(i):
    return (i, 0)


def _rep_map(i):
    return (0, 0)


def _mm0_body(x_ref, w_ref, rw_ref, deg_ref, ua_ref, ub_ref, r_ref, dinv_ref):
    dinv = lax.rsqrt(deg_ref[0, :, 0:1] + deg_ref[1, :, 0:1] + 1.0)
    xb = x_ref[...]
    u = jnp.dot(xb, w_ref[...], preferred_element_type=jnp.float32) * dinv
    ua_ref[...] = u[:, :DH]
    ub_ref[...] = u[:, DH:]
    r_ref[...] = jnp.dot(xb, rw_ref[...], preferred_element_type=jnp.float32)
    dinv_ref[...] = dinv


_mm0 = pl.pallas_call(
    _mm0_body,
    grid=(G,),
    in_specs=[
        pl.BlockSpec((R, NFEAT), _row_map),
        pl.BlockSpec((NFEAT, NHID), _rep_map),
        pl.BlockSpec((NFEAT, NHID), _rep_map),
        pl.BlockSpec((NSC, R, LANES), _part_map),
    ],
    out_specs=[
        pl.BlockSpec((R, DH), _row_map),
        pl.BlockSpec((R, DH), _row_map),
        pl.BlockSpec((R, NHID), _row_map),
        pl.BlockSpec((R, 1), _row_map),
    ],
    out_shape=[
        jax.ShapeDtypeStruct((N, DH), jnp.float32),
        jax.ShapeDtypeStruct((N, DH), jnp.float32),
        jax.ShapeDtypeStruct((N, NHID), jnp.float32),
        jax.ShapeDtypeStruct((N, 1), jnp.float32),
    ],
)


def _mm_body(h_ref, w_ref, dinv_ref, ua_ref, ub_ref):
    u = jnp.dot(
        h_ref[...], w_ref[...], preferred_element_type=jnp.float32
    ) * dinv_ref[...]
    ua_ref[...] = u[:, :DH]
    ub_ref[...] = u[:, DH:]


_mm_mid = pl.pallas_call(
    _mm_body,
    grid=(G,),
    in_specs=[
        pl.BlockSpec((R, NHID), _row_map),
        pl.BlockSpec((NHID, NHID), _rep_map),
        pl.BlockSpec((R, 1), _row_map),
    ],
    out_specs=[
        pl.BlockSpec((R, DH), _row_map),
        pl.BlockSpec((R, DH), _row_map),
    ],
    out_shape=[
        jax.ShapeDtypeStruct((N, DH), jnp.float32),
        jax.ShapeDtypeStruct((N, DH), jnp.float32),
    ],
)


def _mm3_body(h_ref, w_ref, rw_ref, dinv_ref, u_ref, r_ref):
    hb = h_ref[...]
    u_ref[...] = jnp.dot(hb, w_ref[...], preferred_element_type=jnp.float32) * dinv_ref[...]
    r_ref[...] = jnp.dot(hb, rw_ref[...], preferred_element_type=jnp.float32)


_mm3 = pl.pallas_call(
    _mm3_body,
    grid=(G,),
    in_specs=[
        pl.BlockSpec((R, NHID), _row_map),
        pl.BlockSpec((NHID, DC), _rep_map),
        pl.BlockSpec((NHID, DC), _rep_map),
        pl.BlockSpec((R, 1), _row_map),
    ],
    out_specs=[
        pl.BlockSpec((R, DC), _row_map),
        pl.BlockSpec((R, DC), _row_map),
    ],
    out_shape=[
        jax.ShapeDtypeStruct((N, DC), jnp.float32),
        jax.ShapeDtypeStruct((N, DC), jnp.float32),
    ],
)


def _ep0_body(sa_ref, sb_ref, ua_ref, ub_ref, r_ref, dinv_ref, b_ref, rb_ref, o_ref):
    s = jnp.concatenate(
        [sa_ref[0] + sa_ref[1], sb_ref[0] + sb_ref[1]], axis=1
    )
    u = jnp.concatenate([ua_ref[...], ub_ref[...]], axis=1)
    o_ref[...] = jax.nn.relu(
        dinv_ref[...] * (s + u) + r_ref[...] + b_ref[...] + rb_ref[...]
    )


_ep0 = pl.pallas_call(
    _ep0_body,
    grid=(G,),
    in_specs=[
        pl.BlockSpec((NSC, R, DH), _part_map),
        pl.BlockSpec((NSC, R, DH), _part_map),
        pl.BlockSpec((R, DH), _row_map),
        pl.BlockSpec((R, DH), _row_map),
        pl.BlockSpec((R, NHID), _row_map),
        pl.BlockSpec((R, 1), _row_map),
        pl.BlockSpec((1, NHID), _rep_map),
        pl.BlockSpec((1, NHID), _rep_map),
    ],
    out_specs=pl.BlockSpec((R, NHID), _row_map),
    out_shape=jax.ShapeDtypeStruct((N, NHID), jnp.float32),
)


def _ep_mid_body(sa_ref, sb_ref, ua_ref, ub_ref, h_ref, dinv_ref, b_ref, o_ref):
    s = jnp.concatenate(
        [sa_ref[0] + sa_ref[1], sb_ref[0] + sb_ref[1]], axis=1
    )
    u = jnp.concatenate([ua_ref[...], ub_ref[...]], axis=1)
    o_ref[...] = jax.nn.relu(
        dinv_ref[...] * (s + u) + h_ref[...] + b_ref[...]
    )


_ep_mid = pl.pallas_call(
    _ep_mid_body,
    grid=(G,),
    in_specs=[
        pl.BlockSpec((NSC, R, DH), _part_map),
        pl.BlockSpec((NSC, R, DH), _part_map),
        pl.BlockSpec((R, DH), _row_map),
        pl.BlockSpec((R, DH), _row_map),
        pl.BlockSpec((R, NHID), _row_map),
        pl.BlockSpec((R, 1), _row_map),
        pl.BlockSpec((1, NHID), _rep_map),
    ],
    out_specs=pl.BlockSpec((R, NHID), _row_map),
    out_shape=jax.ShapeDtypeStruct((N, NHID), jnp.float32),
)


def _ep3_body(s_ref, u_ref, r_ref, dinv_ref, b_ref, rb_ref, o_ref):
    z = (
        dinv_ref[...] * (s_ref[0] + s_ref[1] + u_ref[...])
        + r_ref[...] + b_ref[...] + rb_ref[...]
    )
    col = lax.broadcasted_iota(jnp.int32, (R, DC), 1)
    z = jnp.where(col < NCLASS, z, -1e30)
    m = jnp.max(z, axis=1, keepdims=True)
    lse = jnp.log(jnp.sum(jnp.exp(z - m), axis=1, keepdims=True)) + m
    o_ref[...] = z - lse


_ep3 = pl.pallas_call(
    _ep3_body,
    grid=(G,),
    in_specs=[
        pl.BlockSpec((NSC, R, DC), _part_map),
        pl.BlockSpec((R, DC), _row_map),
        pl.BlockSpec((R, DC), _row_map),
        pl.BlockSpec((R, 1), _row_map),
        pl.BlockSpec((1, DC), _rep_map),
        pl.BlockSpec((1, DC), _rep_map),
    ],
    out_specs=pl.BlockSpec((R, DC), _row_map),
    out_shape=jax.ShapeDtypeStruct((N, DC), jnp.float32),
)


def kernel(x, edge_index, W0, b0, W1, b1, W2, b2, W3, b3, RW0, Rb0, RW1, Rb1):
    ei = edge_index.astype(jnp.int32)
    src, dst = ei[0], ei[1]

    pad_src = jnp.arange(PADN, dtype=jnp.int32)
    pad_dst = N + pad_src % (NPF - N)
    srcp = jnp.concatenate([src[:EPSC], pad_src, src[EPSC:], pad_src])
    dstp = jnp.concatenate([dst[:EPSC], pad_dst, dst[EPSC:], pad_dst])

    degacc = _get_sc_prep()(dstp)

    def pad_c(a):
        return jnp.zeros(a.shape[:-1] + (DC,), jnp.float32).at[..., :NCLASS].set(a)

    W3p, RW1p = pad_c(W3), pad_c(RW1)
    b3p, Rb1p = pad_c(b3).reshape(1, DC), pad_c(Rb1).reshape(1, DC)
    b0r, Rb0r = b0.reshape(1, NHID), Rb0.reshape(1, NHID)
    b1r, b2r = b1.reshape(1, NHID), b2.reshape(1, NHID)

    sc = _get_sc_edge()

    u0a, u0b, r0, dinv = _mm0(x, W0, RW0, degacc)
    s0a, s0b = sc(u0a, srcp, dstp), sc(u0b, srcp, dstp)
    h1 = _ep0(s0a, s0b, u0a, u0b, r0, dinv, b0r, Rb0r)

    u1a, u1b = _mm_mid(h1, W1, dinv)
    s1a, s1b = sc(u1a, srcp, dstp), sc(u1b, srcp, dstp)
    h2 = _ep_mid(s1a, s1b, u1a, u1b, h1, dinv, b1r)

    u2a, u2b = _mm_mid(h2, W2, dinv)
    s2a, s2b = sc(u2a, srcp, dstp), sc(u2b, srcp, dstp)
    h3 = _ep_mid(s2a, s2b, u2a, u2b, h2, dinv, b2r)

    u3, r3 = _mm3(h3, W3p, RW1p, dinv)
    s3 = sc(u3, srcp, dstp)
    outp = _ep3(s3, u3, r3, dinv, b3p, Rb1p)
    return outp[:, :NCLASS]


# speedup vs baseline: 10.3080x; 1.0375x over previous
import functools

import jax
import jax.numpy as jnp
from jax import lax
from jax.experimental import pallas as pl
from jax.experimental.pallas import tpu as pltpu
from jax.experimental.pallas import tpu_sc as plsc

N = 10000
E = 160000
NFEAT = 256
NHID = 256
NCLASS = 40
DH = 128
DC = 128

NSC = 2
NSUB = 16
EPSC = E // NSC
K = 80
ETS = 5120
EPAD = NSUB * ETS
PADN = EPAD - EPSC
NCHUNK = ETS // K
DEPTH = 4
LANES = 16
ZR = 32

NPF = 10240
ROWS_OUT = NPF // NSUB

R = 1000
G = N // R


@functools.cache
def _get_mesh():
    return plsc.VectorSubcoreMesh(
        core_axis_name="c", subcore_axis_name="s", num_cores=NSC, num_subcores=NSUB
    )


def _sc_prep_body(dstp_hbm, degacc_hbm, dbuf, ones, zbuf, acc):
    c = lax.axis_index("c")
    s = lax.axis_index("s")

    def fill_ones(j, _):
        ones[j, :] = jnp.full((LANES,), 1.0, jnp.float32)
        zbuf[j, :] = jnp.zeros((LANES,), jnp.float32)
        return 0
    lax.fori_loop(0, K, fill_ones, 0)

    for q in range(ROWS_OUT // K):
        pltpu.sync_copy(zbuf, acc.at[pl.ds(s * ROWS_OUT + q * K, K)])
    plsc.subcore_barrier()

    def chunk(i, _):
        base = c * EPAD + s * ETS + i * K
        pltpu.sync_copy(dstp_hbm.at[pl.ds(base, K)], dbuf)
        pltpu.sync_copy(ones, acc.at[dbuf], add=True)
        return 0
    lax.fori_loop(0, NCHUNK, chunk, 0)
    plsc.subcore_barrier()

    pltpu.sync_copy(
        acc.at[pl.ds(s * ROWS_OUT, ROWS_OUT)],
        degacc_hbm.at[c, pl.ds(s * ROWS_OUT, ROWS_OUT)],
    )


@functools.cache
def _get_sc_prep():
    return pl.kernel(
        _sc_prep_body,
        out_type=jax.ShapeDtypeStruct((NSC, NPF, LANES), jnp.float32),
        mesh=_get_mesh(),
        scratch_types=[
            pltpu.VMEM((K,), jnp.int32),
            pltpu.VMEM((K, LANES), jnp.float32),
            pltpu.VMEM((K, LANES), jnp.float32),
            pltpu.VMEM_SHARED((NPF, LANES), jnp.float32),
        ],
    )


def _zfill(zrow):
    def zf(j, _):
        for kk in range(DH // LANES):
            zrow[j, pl.ds(kk * LANES, LANES)] = jnp.zeros((LANES,), jnp.float32)
        return 0
    lax.fori_loop(0, ZR, zf, 0)


def _edge_phase(u_hbm, out_hbm, srcp_hbm, dstp_hbm, ibs, dbs, gbs, zrow, acc,
                sems, c, s):
    def zero(q, _):
        pltpu.sync_copy(zrow, acc.at[pl.ds(s * ROWS_OUT + q * ZR, ZR)])
        return 0
    lax.fori_loop(0, ROWS_OUT // ZR, zero, 0)
    plsc.subcore_barrier()

    def quad(i, _):
        e0 = c * EPAD + s * ETS + i * (DEPTH * K)
        cps = []
        for j in range(DEPTH):
            pltpu.sync_copy(srcp_hbm.at[pl.ds(e0 + j * K, K)], ibs[j])
            pltpu.sync_copy(dstp_hbm.at[pl.ds(e0 + j * K, K)], dbs[j])
            cps.append(pltpu.async_copy(u_hbm.at[ibs[j]], gbs[j], sems[j]))
        for j in range(DEPTH):
            cps[j].wait()
            pltpu.sync_copy(gbs[j], acc.at[dbs[j]], add=True)
        return 0
    lax.fori_loop(0, NCHUNK // DEPTH, quad, 0)
    plsc.subcore_barrier()

    pltpu.sync_copy(
        acc.at[pl.ds(s * ROWS_OUT, ROWS_OUT)],
        out_hbm.at[c, pl.ds(s * ROWS_OUT, ROWS_OUT)],
    )


def _sc_edge_body(u_hbm, srcp_hbm, dstp_hbm, s_hbm,
                  ib0, ib1, ib2, ib3, db0, db1, db2, db3,
                  gb0, gb1, gb2, gb3, zrow, acc, sm0, sm1, sm2, sm3):
    c = lax.axis_index("c")
    s = lax.axis_index("s")
    ibs, dbs = (ib0, ib1, ib2, ib3), (db0, db1, db2, db3)
    gbs, sems = (gb0, gb1, gb2, gb3), (sm0, sm1, sm2, sm3)
    _zfill(zrow)
    _edge_phase(u_hbm, s_hbm, srcp_hbm, dstp_hbm, ibs, dbs, gbs, zrow, acc, sems, c, s)


def _edge_scratch():
    return (
        [pltpu.VMEM((K,), jnp.int32)] * 8
        + [pltpu.VMEM((K, DH), jnp.float32)] * 4
        + [pltpu.VMEM((ZR, DH), jnp.float32)]
        + [pltpu.VMEM_SHARED((NPF, DH), jnp.float32)]
        + [pltpu.SemaphoreType.DMA] * 4
    )


@functools.cache
def _get_sc_edge():
    return pl.kernel(
        _sc_edge_body,
        out_type=jax.ShapeDtypeStruct((NSC, NPF, DH), jnp.float32),
        mesh=_get_mesh(),
        scratch_types=_edge_scratch(),
    )


def _part_map(i):
    return (0, i, 0)


def _row_map(i):
    return (i, 0)


def _rep_map(i):
    return (0, 0)


def _mm_pre_body(x_ref, w_ref, rw_ref, hw_ref, r_ref):
    xb = x_ref[...]
    hw_ref[...] = jnp.dot(xb, w_ref[...], preferred_element_type=jnp.float32)
    r_ref[...] = jnp.dot(xb, rw_ref[...], preferred_element_type=jnp.float32)


_mm_pre = pl.pallas_call(
    _mm_pre_body,
    grid=(G,),
    in_specs=[
        pl.BlockSpec((R, NFEAT), _row_map),
        pl.BlockSpec((NFEAT, NHID), _rep_map),
        pl.BlockSpec((NFEAT, NHID), _rep_map),
    ],
    out_specs=[
        pl.BlockSpec((R, NHID), _row_map),
        pl.BlockSpec((R, NHID), _row_map),
    ],
    out_shape=[
        jax.ShapeDtypeStruct((N, NHID), jnp.float32),
        jax.ShapeDtypeStruct((N, NHID), jnp.float32),
    ],
)


def _scale0_body(hw_ref, deg_ref, ua_ref, ub_ref, dinv_ref):
    dinv = lax.rsqrt(deg_ref[0, :, 0:1] + deg_ref[1, :, 0:1] + 1.0)
    u = hw_ref[...] * dinv
    ua_ref[...] = u[:, :DH]
    ub_ref[...] = u[:, DH:]
    dinv_ref[...] = dinv


_scale0 = pl.pallas_call(
    _scale0_body,
    grid=(G,),
    in_specs=[
        pl.BlockSpec((R, NHID), _row_map),
        pl.BlockSpec((NSC, R, LANES), _part_map),
    ],
    out_specs=[
        pl.BlockSpec((R, DH), _row_map),
        pl.BlockSpec((R, DH), _row_map),
        pl.BlockSpec((R, 1), _row_map),
    ],
    out_shape=[
        jax.ShapeDtypeStruct((N, DH), jnp.float32),
        jax.ShapeDtypeStruct((N, DH), jnp.float32),
        jax.ShapeDtypeStruct((N, 1), jnp.float32),
    ],
)


def _epmm_body(sa_ref, sb_ref, ua_ref, ub_ref, res_ref, dinv_ref, b_ref,
               w_ref, h_ref, va_ref, vb_ref):
    s = jnp.concatenate(
        [sa_ref[0] + sa_ref[1], sb_ref[0] + sb_ref[1]], axis=1
    )
    u = jnp.concatenate([ua_ref[...], ub_ref[...]], axis=1)
    h = jax.nn.relu(dinv_ref[...] * (s + u) + res_ref[...] + b_ref[...])
    h_ref[...] = h
    v = jnp.dot(h, w_ref[...], preferred_element_type=jnp.float32) * dinv_ref[...]
    va_ref[...] = v[:, :DH]
    vb_ref[...] = v[:, DH:]


_epmm = pl.pallas_call(
    _epmm_body,
    grid=(G,),
    in_specs=[
        pl.BlockSpec((NSC, R, DH), _part_map),
        pl.BlockSpec((NSC, R, DH), _part_map),
        pl.BlockSpec((R, DH), _row_map),
        pl.BlockSpec((R, DH), _row_map),
        pl.BlockSpec((R, NHID), _row_map),
        pl.BlockSpec((R, 1), _row_map),
        pl.BlockSpec((1, NHID), _rep_map),
        pl.BlockSpec((NHID, NHID), _rep_map),
    ],
    out_specs=[
        pl.BlockSpec((R, NHID), _row_map),
        pl.BlockSpec((R, DH), _row_map),
        pl.BlockSpec((R, DH), _row_map),
    ],
    out_shape=[
        jax.ShapeDtypeStruct((N, NHID), jnp.float32),
        jax.ShapeDtypeStruct((N, DH), jnp.float32),
        jax.ShapeDtypeStruct((N, DH), jnp.float32),
    ],
)


def _epmm3_body(sa_ref, sb_ref, ua_ref, ub_ref, h_ref, dinv_ref, b_ref,
                w_ref, rw_ref, u_ref, r_ref):
    s = jnp.concatenate(
        [sa_ref[0] + sa_ref[1], sb_ref[0] + sb_ref[1]], axis=1
    )
    u = jnp.concatenate([ua_ref[...], ub_ref[...]], axis=1)
    h = jax.nn.relu(dinv_ref[...] * (s + u) + h_ref[...] + b_ref[...])
    u_ref[...] = jnp.dot(h, w_ref[...], preferred_element_type=jnp.float32) * dinv_ref[...]
    r_ref[...] = jnp.dot(h, rw_ref[...], preferred_element_type=jnp.float32)


_epmm3 = pl.pallas_call(
    _epmm3_body,
    grid=(G,),
    in_specs=[
        pl.BlockSpec((NSC, R, DH), _part_map),
        pl.BlockSpec((NSC, R, DH), _part_map),
        pl.BlockSpec((R, DH), _row_map),
        pl.BlockSpec((R, DH), _row_map),
        pl.BlockSpec((R, NHID), _row_map),
        pl.BlockSpec((R, 1), _row_map),
        pl.BlockSpec((1, NHID), _rep_map),
        pl.BlockSpec((NHID, DC), _rep_map),
        pl.BlockSpec((NHID, DC), _rep_map),
    ],
    out_specs=[
        pl.BlockSpec((R, DC), _row_map),
        pl.BlockSpec((R, DC), _row_map),
    ],
    out_shape=[
        jax.ShapeDtypeStruct((N, DC), jnp.float32),
        jax.ShapeDtypeStruct((N, DC), jnp.float32),
    ],
)


def _ep3_body(s_ref, u_ref, r_ref, dinv_ref, b_ref, rb_ref, o_ref):
    z = (
        dinv_ref[...] * (s_ref[0] + s_ref[1] + u_ref[...])
        + r_ref[...] + b_ref[...] + rb_ref[...]
    )
    col = lax.broadcasted_iota(jnp.int32, (R, DC), 1)
    z = jnp.where(col < NCLASS, z, -1e30)
    m = jnp.max(z, axis=1, keepdims=True)
    lse = jnp.log(jnp.sum(jnp.exp(z - m), axis=1, keepdims=True)) + m
    o_ref[...] = z - lse


_ep3 = pl.pallas_call(
    _ep3_body,
    grid=(G,),
    in_specs=[
        pl.BlockSpec((NSC, R, DC), _part_map),
        pl.BlockSpec((R, DC), _row_map),
        pl.BlockSpec((R, DC), _row_map),
        pl.BlockSpec((R, 1), _row_map),
        pl.BlockSpec((1, DC), _rep_map),
        pl.BlockSpec((1, DC), _rep_map),
    ],
    out_specs=pl.BlockSpec((R, DC), _row_map),
    out_shape=jax.ShapeDtypeStruct((N, DC), jnp.float32),
)


def kernel(x, edge_index, W0, b0, W1, b1, W2, b2, W3, b3, RW0, Rb0, RW1, Rb1):
    ei = edge_index.astype(jnp.int32)
    src, dst = ei[0], ei[1]

    pad_src = jnp.arange(PADN, dtype=jnp.int32)
    pad_dst = N + pad_src % (NPF - N)
    srcp = jnp.concatenate([src[:EPSC], pad_src, src[EPSC:], pad_src])
    dstp = jnp.concatenate([dst[:EPSC], pad_dst, dst[EPSC:], pad_dst])

    degacc = _get_sc_prep()(dstp)

    def pad_c(a):
        return jnp.zeros(a.shape[:-1] + (DC,), jnp.float32).at[..., :NCLASS].set(a)

    W3p, RW1p = pad_c(W3), pad_c(RW1)
    b3p, Rb1p = pad_c(b3).reshape(1, DC), pad_c(Rb1).reshape(1, DC)
    b0r = (b0 + Rb0).reshape(1, NHID)
    b1r, b2r = b1.reshape(1, NHID), b2.reshape(1, NHID)

    sc = _get_sc_edge()

    hw0, r0 = _mm_pre(x, W0, RW0)
    u0a, u0b, dinv = _scale0(hw0, degacc)
    s0a, s0b = sc(u0a, srcp, dstp), sc(u0b, srcp, dstp)
    h1, u1a, u1b = _epmm(s0a, s0b, u0a, u0b, r0, dinv, b0r, W1)

    s1a, s1b = sc(u1a, srcp, dstp), sc(u1b, srcp, dstp)
    h2, u2a, u2b = _epmm(s1a, s1b, u1a, u1b, h1, dinv, b1r, W2)

    s2a, s2b = sc(u2a, srcp, dstp), sc(u2b, srcp, dstp)
    u3, r3 = _epmm3(s2a, s2b, u2a, u2b, h2, dinv, b2r, W3p, RW1p)

    s3 = sc(u3, srcp, dstp)
    outp = _ep3(s3, u3, r3, dinv, b3p, Rb1p)
    return outp[:, :NCLASS]
